# Initial kernel scaffold; baseline (speedup 1.0000x reference)
import jax
import jax.numpy as jnp
from jax.experimental import pallas as pl


def kernel(x, tables):
    raise NotImplementedError("write your pallas kernel here")


# speedup vs baseline: 2.7162x; 2.7162x over previous
import functools

import jax
import jax.numpy as jnp
from jax import lax
from jax.experimental import pallas as pl
from jax.experimental.pallas import tpu as pltpu
from jax.experimental.pallas import tpu_sc as plsc

NUM_CORES = 2
NUM_SUBCORES = 16
NUM_WORKERS = NUM_CORES * NUM_SUBCORES
LANES = 16

CHUNK = 64
HIDDEN = 128
NFEAT = 9
IDX_PER_CHUNK = CHUNK * NFEAT


def _sc_body(n_rows, n_chunks, k_per_worker,
             xflat_hbm, off_hbm, tbl_hbm, out_hbm,
             idx_v, off_v, rows_v, acc_v, sem):
    wid = lax.axis_index("s") * NUM_CORES + lax.axis_index("c")

    pltpu.sync_copy(off_hbm, off_v)

    def chunk_body(k, carry):
        c = wid + NUM_WORKERS * k
        base = c * CHUNK

        @pl.when(base < n_rows)
        def _():
            pltpu.sync_copy(xflat_hbm.at[pl.ds(c * IDX_PER_CHUNK, IDX_PER_CHUNK)],
                            idx_v)

            def add_off(j, _):
                sl = pl.ds(j * LANES, LANES)
                idx_v[sl] = idx_v[sl] + off_v[sl]
                return 0

            lax.fori_loop(0, IDX_PER_CHUNK // LANES, add_off, 0)

            descs = []
            done = 0
            while done < IDX_PER_CHUNK:
                step = min(128, IDX_PER_CHUNK - done)
                descs.append(
                    pltpu.async_copy(
                        tbl_hbm.at[idx_v.at[pl.ds(done, step)]],
                        rows_v.at[pl.ds(done, step)],
                        sem,
                    ))
                done += step
            for d in descs:
                d.wait()

            def reduce_row(r, _):
                for cc in range(HIDDEN // LANES):
                    sl = pl.ds(cc * LANES, LANES)
                    s = rows_v[r * NFEAT, sl]
                    for i in range(1, NFEAT):
                        s = s + rows_v[r * NFEAT + i, sl]
                    acc_v[r, sl] = s
                return 0

            lax.fori_loop(0, CHUNK, reduce_row, 0)

            full = base + CHUNK <= n_rows

            @pl.when(full)
            def _():
                pltpu.sync_copy(acc_v, out_hbm.at[pl.ds(base, CHUNK)])

            tail = n_rows % CHUNK
            if tail:
                @pl.when(jnp.logical_not(full))
                def _():
                    pltpu.sync_copy(acc_v.at[pl.ds(0, tail)],
                                    out_hbm.at[pl.ds(base, tail)])

        return carry

    lax.fori_loop(0, k_per_worker, chunk_body, 0)


def kernel(x, tables):
    n_rows = x.shape[0]
    n_chunks = -(-n_rows // CHUNK)
    k_per_worker = -(-n_chunks // NUM_WORKERS)

    tbl = tables.reshape(NFEAT * tables.shape[1], tables.shape[2])
    xflat = x.astype(jnp.int32).reshape(-1)
    pad = n_chunks * IDX_PER_CHUNK - xflat.shape[0]
    if pad:
        xflat = jnp.concatenate([xflat, jnp.zeros((pad,), jnp.int32)])
    off = jnp.tile(jnp.arange(NFEAT, dtype=jnp.int32) * tables.shape[1], CHUNK)

    mesh = plsc.VectorSubcoreMesh(core_axis_name="c", subcore_axis_name="s")
    body = functools.partial(_sc_body, n_rows, n_chunks, k_per_worker)
    run = pl.kernel(
        body,
        mesh=mesh,
        out_type=jax.ShapeDtypeStruct((n_rows, HIDDEN), jnp.float32),
        scratch_types=[
            pltpu.VMEM((IDX_PER_CHUNK,), jnp.int32),
            pltpu.VMEM((IDX_PER_CHUNK,), jnp.int32),
            pltpu.VMEM((IDX_PER_CHUNK, HIDDEN), jnp.float32),
            pltpu.VMEM((CHUNK, HIDDEN), jnp.float32),
            pltpu.SemaphoreType.DMA,
        ],
    )
    return run(xflat, off, tbl)


# speedup vs baseline: 3.8286x; 1.4095x over previous
import functools

import jax
import jax.numpy as jnp
from jax import lax
from jax.experimental import pallas as pl
from jax.experimental.pallas import tpu as pltpu
from jax.experimental.pallas import tpu_sc as plsc

NUM_CORES = 2
NUM_SUBCORES = 16
NUM_WORKERS = NUM_CORES * NUM_SUBCORES
LANES = 16
NBUF = 2

CHUNK = 32
HIDDEN = 128
NFEAT = 9
IDX_PER_CHUNK = CHUNK * NFEAT


def _gather_descs(tbl_hbm, idx_v, rows_v, sem):
    descs = []
    done = 0
    while done < IDX_PER_CHUNK:
        step = min(128, IDX_PER_CHUNK - done)
        descs.append(
            pltpu.make_async_copy(
                tbl_hbm.at[idx_v.at[pl.ds(done, step)]],
                rows_v.at[pl.ds(done, step)],
                sem,
            ))
        done += step
    return descs


def _sc_body(n_rows, k_lim_cap,
             xflat_hbm, off_hbm, tbl_hbm, out_hbm,
             idx0, idx1, rows0, rows1, acc0, acc1, off_v,
             gsem0, gsem1, osem0, osem1):
    wid = lax.axis_index("s") * NUM_CORES + lax.axis_index("c")
    idx = (idx0, idx1)
    rows = (rows0, rows1)
    acc = (acc0, acc1)
    gsem = (gsem0, gsem1)
    osem = (osem0, osem1)

    n_chunks = n_rows // CHUNK
    k_lim = (n_chunks - 1 - wid) // NUM_WORKERS + 1

    pltpu.sync_copy(off_hbm, off_v)

    def issue(k, b):
        c = wid + NUM_WORKERS * k
        pltpu.sync_copy(
            xflat_hbm.at[pl.ds(c * IDX_PER_CHUNK, IDX_PER_CHUNK)], idx[b])

        def add_off(j, _):
            sl = pl.ds(j * LANES, LANES)
            idx[b][sl] = idx[b][sl] + off_v[sl]
            return 0

        lax.fori_loop(0, IDX_PER_CHUNK // LANES, add_off, 0)
        for d in _gather_descs(tbl_hbm, idx[b], rows[b], gsem[b]):
            d.start()

    for b in range(NBUF):
        @pl.when(b < k_lim)
        def _(b=b):
            issue(b, b)

    def pair_body(g, _):
        for b in range(NBUF):
            k = NBUF * g + b

            @pl.when(k < k_lim)
            def _(k=k, b=b):
                c = wid + NUM_WORKERS * k
                base = c * CHUNK
                for d in _gather_descs(tbl_hbm, idx[b], rows[b], gsem[b]):
                    d.wait()
                @pl.when(k >= NBUF)
                def _():
                    pltpu.make_async_copy(
                        acc[b], out_hbm.at[pl.ds(0, CHUNK)], osem[b]).wait()

                def reduce_row(r, _):
                    for cc in range(HIDDEN // LANES):
                        sl = pl.ds(cc * LANES, LANES)
                        s = rows[b][r * NFEAT, sl]
                        for i in range(1, NFEAT):
                            s = s + rows[b][r * NFEAT + i, sl]
                        acc[b][r, sl] = s
                    return 0

                lax.fori_loop(0, CHUNK, reduce_row, 0)

                pltpu.make_async_copy(
                    acc[b], out_hbm.at[pl.ds(base, CHUNK)], osem[b]).start()

                @pl.when(k + NBUF < k_lim)
                def _():
                    issue(k + NBUF, b)

        return 0

    lax.fori_loop(0, (k_lim_cap + NBUF - 1) // NBUF, pair_body, 0)

    for b in range(NBUF):
        @pl.when(k_lim >= b + 1)
        def _(b=b):
            pltpu.make_async_copy(
                acc[b], out_hbm.at[pl.ds(0, CHUNK)], osem[b]).wait()


def kernel(x, tables):
    n_rows = x.shape[0]
    n_chunks = n_rows // CHUNK
    assert n_chunks * CHUNK == n_rows
    k_lim_cap = -(-n_chunks // NUM_WORKERS)

    tbl = tables.reshape(NFEAT * tables.shape[1], tables.shape[2])
    xflat = x.astype(jnp.int32).reshape(-1)
    off = jnp.tile(jnp.arange(NFEAT, dtype=jnp.int32) * tables.shape[1], CHUNK)

    mesh = plsc.VectorSubcoreMesh(core_axis_name="c", subcore_axis_name="s")
    body = functools.partial(_sc_body, n_rows, k_lim_cap)
    run = pl.kernel(
        body,
        mesh=mesh,
        out_type=jax.ShapeDtypeStruct((n_rows, HIDDEN), jnp.float32),
        scratch_types=[
            pltpu.VMEM((IDX_PER_CHUNK,), jnp.int32),
            pltpu.VMEM((IDX_PER_CHUNK,), jnp.int32),
            pltpu.VMEM((IDX_PER_CHUNK, HIDDEN), jnp.float32),
            pltpu.VMEM((IDX_PER_CHUNK, HIDDEN), jnp.float32),
            pltpu.VMEM((CHUNK, HIDDEN), jnp.float32),
            pltpu.VMEM((CHUNK, HIDDEN), jnp.float32),
            pltpu.VMEM((IDX_PER_CHUNK,), jnp.int32),
            pltpu.SemaphoreType.DMA,
            pltpu.SemaphoreType.DMA,
            pltpu.SemaphoreType.DMA,
            pltpu.SemaphoreType.DMA,
        ],
    )
    return run(xflat, off, tbl)


# speedup vs baseline: 4.9191x; 1.2848x over previous
import functools

import jax
import jax.numpy as jnp
from jax import lax
from jax.experimental import pallas as pl
from jax.experimental.pallas import tpu as pltpu
from jax.experimental.pallas import tpu_sc as plsc

NUM_CORES = 2
NUM_SUBCORES = 16
NUM_WORKERS = NUM_CORES * NUM_SUBCORES
LANES = 16
NBUF = 3

CHUNK = 200
HIDDEN = 128
NFEAT = 9
IDX_PER_CHUNK = CHUNK * NFEAT


def _gather_descs(tbl_hbm, idx_v, acc_v, sem):
    descs = []
    for i in range(NFEAT):
        done = 0
        while done < CHUNK:
            step = min(128, CHUNK - done)
            descs.append(
                pltpu.make_async_copy(
                    tbl_hbm.at[idx_v.at[pl.ds(i * CHUNK + done, step)]],
                    acc_v.at[pl.ds(done, step)],
                    sem,
                ))
            done += step
    return descs


def _sc_body(n_rows, k_lim_cap,
             xoff_hbm, tbl_hbm, out_hbm,
             idx0, idx1, idx2, acc0, acc1, acc2,
             gsem0, gsem1, gsem2, osem0, osem1, osem2):
    wid = lax.axis_index("s") * NUM_CORES + lax.axis_index("c")
    idx = (idx0, idx1, idx2)
    acc = (acc0, acc1, acc2)
    gsem = (gsem0, gsem1, gsem2)
    osem = (osem0, osem1, osem2)

    n_chunks = n_rows // CHUNK
    k_lim = (n_chunks - 1 - wid) // NUM_WORKERS + 1

    def issue(k, b):
        c = wid + NUM_WORKERS * k
        base = c * CHUNK
        @pl.when(k >= NBUF)
        def _():
            pltpu.make_async_copy(
                acc[b], out_hbm.at[pl.ds(0, CHUNK)], osem[b]).wait()

        zero = jnp.zeros((LANES,), jnp.float32)

        def zrow(r, _):
            for cc in range(HIDDEN // LANES):
                acc[b][r, pl.ds(cc * LANES, LANES)] = zero
            return 0

        lax.fori_loop(0, CHUNK, zrow, 0)

        for i in range(NFEAT):
            pltpu.sync_copy(
                xoff_hbm.at[pl.ds(i * n_rows + base, CHUNK)],
                idx[b].at[pl.ds(i * CHUNK, CHUNK)])
        for d in _gather_descs(tbl_hbm, idx[b], acc[b], gsem[b]):
            d.start(add=True)

    for b in range(NBUF - 1):
        @pl.when(b < k_lim)
        def _(b=b):
            issue(b, b)

    def group_body(g, _):
        for j in range(NBUF):
            k = NBUF * g + j

            @pl.when(k < k_lim)
            def _(k=k, j=j):
                b = j
                c = wid + NUM_WORKERS * k
                base = c * CHUNK
                for d in _gather_descs(tbl_hbm, idx[b], acc[b], gsem[b]):
                    d.wait()
                pltpu.make_async_copy(
                    acc[b], out_hbm.at[pl.ds(base, CHUNK)], osem[b]).start()

                @pl.when(k + NBUF - 1 < k_lim)
                def _():
                    issue(k + NBUF - 1, (j + NBUF - 1) % NBUF)

        return 0

    lax.fori_loop(0, (k_lim_cap + NBUF - 1) // NBUF, group_body, 0)

    for b in range(NBUF):
        @pl.when(k_lim > b)
        def _(b=b):
            pltpu.make_async_copy(
                acc[b], out_hbm.at[pl.ds(0, CHUNK)], osem[b]).wait()


def kernel(x, tables):
    n_rows = x.shape[0]
    n_chunks = n_rows // CHUNK
    assert n_chunks * CHUNK == n_rows
    k_lim_cap = -(-n_chunks // NUM_WORKERS)

    vocab = tables.shape[1]
    tbl = tables.reshape(NFEAT * vocab, tables.shape[2])
    xoff = (x.astype(jnp.int32) +
            jnp.arange(NFEAT, dtype=jnp.int32)[None, :] * vocab).T.reshape(-1)

    mesh = plsc.VectorSubcoreMesh(core_axis_name="c", subcore_axis_name="s")
    body = functools.partial(_sc_body, n_rows, k_lim_cap)
    run = pl.kernel(
        body,
        mesh=mesh,
        out_type=jax.ShapeDtypeStruct((n_rows, HIDDEN), jnp.float32),
        scratch_types=(
            [pltpu.VMEM((IDX_PER_CHUNK,), jnp.int32)] * NBUF
            + [pltpu.VMEM((CHUNK, HIDDEN), jnp.float32)] * NBUF
            + [pltpu.SemaphoreType.DMA] * (2 * NBUF)
        ),
    )
    return run(xoff, tbl)


# speedup vs baseline: 5.0599x; 1.0286x over previous
import functools

import jax
import jax.numpy as jnp
from jax import lax
from jax.experimental import pallas as pl
from jax.experimental.pallas import tpu as pltpu
from jax.experimental.pallas import tpu_sc as plsc

NUM_CORES = 2
NUM_SUBCORES = 16
NUM_WORKERS = NUM_CORES * NUM_SUBCORES
LANES = 16
NBUF = 3

CHUNK = 200
HIDDEN = 128
VOCAB = 100
NPAIR = 4
NGRP = NPAIR + 1
IDX_PER_CHUNK = CHUNK * NGRP


def _build_body(tref, oref):
    j = pl.program_id(0)
    p = j // VOCAB
    a = j % VOCAB
    oref[0, 0] = tref[2 * p, a][None, :] + tref[2 * p + 1]


def _build_pair_tables(tables):
    out = pl.pallas_call(
        _build_body,
        grid=(NPAIR * VOCAB,),
        in_specs=[pl.BlockSpec(tables.shape, lambda j: (0, 0, 0))],
        out_specs=pl.BlockSpec((1, 1, VOCAB, HIDDEN),
                               lambda j: (j // VOCAB, j % VOCAB, 0, 0)),
        out_shape=jax.ShapeDtypeStruct((NPAIR, VOCAB, VOCAB, HIDDEN),
                                       jnp.float32),
    )(tables)
    return out.reshape(NPAIR * VOCAB * VOCAB, HIDDEN)


def _gather_descs(tbig_hbm, tbl8_hbm, idx_v, acc_v, sem):
    descs = []
    for i in range(NGRP):
        src = tbig_hbm if i < NPAIR else tbl8_hbm
        done = 0
        while done < CHUNK:
            step = min(128, CHUNK - done)
            descs.append(
                pltpu.make_async_copy(
                    src.at[idx_v.at[pl.ds(i * CHUNK + done, step)]],
                    acc_v.at[pl.ds(done, step)],
                    sem,
                ))
            done += step
    return descs


def _sc_body(n_rows, k_lim_cap,
             xoff_hbm, tbig_hbm, tbl8_hbm, out_hbm,
             idx0, idx1, idx2, acc0, acc1, acc2,
             gsem0, gsem1, gsem2, osem0, osem1, osem2):
    wid = lax.axis_index("s") * NUM_CORES + lax.axis_index("c")
    idx = (idx0, idx1, idx2)
    acc = (acc0, acc1, acc2)
    gsem = (gsem0, gsem1, gsem2)
    osem = (osem0, osem1, osem2)

    n_chunks = n_rows // CHUNK
    k_lim = (n_chunks - 1 - wid) // NUM_WORKERS + 1

    def issue(k, b):
        c = wid + NUM_WORKERS * k
        base = c * CHUNK
        @pl.when(k >= NBUF)
        def _():
            pltpu.make_async_copy(
                acc[b], out_hbm.at[pl.ds(0, CHUNK)], osem[b]).wait()

        zero = jnp.zeros((LANES,), jnp.float32)

        def zrow(r, _):
            for cc in range(HIDDEN // LANES):
                acc[b][r, pl.ds(cc * LANES, LANES)] = zero
            return 0

        lax.fori_loop(0, CHUNK, zrow, 0)

        for i in range(NGRP):
            pltpu.sync_copy(
                xoff_hbm.at[pl.ds(i * n_rows + base, CHUNK)],
                idx[b].at[pl.ds(i * CHUNK, CHUNK)])
        for d in _gather_descs(tbig_hbm, tbl8_hbm, idx[b], acc[b], gsem[b]):
            d.start(add=True)

    for b in range(NBUF - 1):
        @pl.when(b < k_lim)
        def _(b=b):
            issue(b, b)

    def group_body(g, _):
        for j in range(NBUF):
            k = NBUF * g + j

            @pl.when(k < k_lim)
            def _(k=k, j=j):
                b = j
                c = wid + NUM_WORKERS * k
                base = c * CHUNK
                for d in _gather_descs(tbig_hbm, tbl8_hbm, idx[b], acc[b],
                                       gsem[b]):
                    d.wait()
                pltpu.make_async_copy(
                    acc[b], out_hbm.at[pl.ds(base, CHUNK)], osem[b]).start()

                @pl.when(k + NBUF - 1 < k_lim)
                def _():
                    issue(k + NBUF - 1, (j + NBUF - 1) % NBUF)

        return 0

    lax.fori_loop(0, (k_lim_cap + NBUF - 1) // NBUF, group_body, 0)

    for b in range(NBUF):
        @pl.when(k_lim > b)
        def _(b=b):
            pltpu.make_async_copy(
                acc[b], out_hbm.at[pl.ds(0, CHUNK)], osem[b]).wait()


def kernel(x, tables):
    n_rows = x.shape[0]
    n_chunks = n_rows // CHUNK
    assert n_chunks * CHUNK == n_rows
    k_lim_cap = -(-n_chunks // NUM_WORKERS)

    tbig = _build_pair_tables(tables)
    tbl8 = tables[2 * NPAIR]

    xi = x.astype(jnp.int32)
    pair_idx = (VOCAB * xi[:, 0:2 * NPAIR:2] + xi[:, 1:2 * NPAIR:2]
                + jnp.arange(NPAIR, dtype=jnp.int32)[None, :] * (VOCAB * VOCAB))
    xoff = jnp.concatenate([pair_idx.T, xi[:, 2 * NPAIR][None, :]], axis=0)
    xoff = xoff.reshape(-1)

    mesh = plsc.VectorSubcoreMesh(core_axis_name="c", subcore_axis_name="s")
    body = functools.partial(_sc_body, n_rows, k_lim_cap)
    run = pl.kernel(
        body,
        mesh=mesh,
        out_type=jax.ShapeDtypeStruct((n_rows, HIDDEN), jnp.float32),
        scratch_types=(
            [pltpu.VMEM((IDX_PER_CHUNK,), jnp.int32)] * NBUF
            + [pltpu.VMEM((CHUNK, HIDDEN), jnp.float32)] * NBUF
            + [pltpu.SemaphoreType.DMA] * (2 * NBUF)
        ),
    )
    return run(xoff, tbig, tbl8)


# speedup vs baseline: 7.4535x; 1.4730x over previous
import functools

import jax
import jax.numpy as jnp
from jax import lax
from jax.experimental import pallas as pl
from jax.experimental.pallas import tpu as pltpu
from jax.experimental.pallas import tpu_sc as plsc

NUM_CORES = 2
NUM_SUBCORES = 16
NUM_WORKERS = NUM_CORES * NUM_SUBCORES
LANES = 16
NBUF = 3

CHUNK = 200
HIDDEN = 128
VOCAB = 100
NPAIR = 4
NGRP = NPAIR + 1
IDX_PER_CHUNK = CHUNK * NGRP


def _build_body(tref, oref):
    p = pl.program_id(0)
    b_rows = tref[2 * p + 1]

    def row(a, _):
        oref[0, a] = tref[2 * p, a][None, :] + b_rows
        return 0

    lax.fori_loop(0, VOCAB, row, 0)


def _build_pair_tables(tables):
    out = pl.pallas_call(
        _build_body,
        grid=(NPAIR,),
        in_specs=[pl.BlockSpec(tables.shape, lambda p: (0, 0, 0))],
        out_specs=pl.BlockSpec((1, VOCAB, VOCAB, HIDDEN),
                               lambda p: (p, 0, 0, 0)),
        out_shape=jax.ShapeDtypeStruct((NPAIR, VOCAB, VOCAB, HIDDEN),
                                       jnp.float32),
    )(tables)
    return out.reshape(NPAIR * VOCAB * VOCAB, HIDDEN)


def _gather_descs(tbig_hbm, tbl8_hbm, idx_v, acc_v, sem):
    descs = []
    for i in range(NGRP):
        src = tbig_hbm if i < NPAIR else tbl8_hbm
        done = 0
        while done < CHUNK:
            step = min(128, CHUNK - done)
            descs.append(
                pltpu.make_async_copy(
                    src.at[idx_v.at[pl.ds(i * CHUNK + done, step)]],
                    acc_v.at[pl.ds(done, step)],
                    sem,
                ))
            done += step
    return descs


def _sc_body(n_rows, k_lim_cap,
             xoff_hbm, tbig_hbm, tbl8_hbm, out_hbm,
             idx0, idx1, idx2, acc0, acc1, acc2,
             gsem0, gsem1, gsem2, osem0, osem1, osem2):
    wid = lax.axis_index("s") * NUM_CORES + lax.axis_index("c")
    idx = (idx0, idx1, idx2)
    acc = (acc0, acc1, acc2)
    gsem = (gsem0, gsem1, gsem2)
    osem = (osem0, osem1, osem2)

    n_chunks = n_rows // CHUNK
    k_lim = (n_chunks - 1 - wid) // NUM_WORKERS + 1

    def issue(k, b):
        c = wid + NUM_WORKERS * k
        base = c * CHUNK
        @pl.when(k >= NBUF)
        def _():
            pltpu.make_async_copy(
                acc[b], out_hbm.at[pl.ds(0, CHUNK)], osem[b]).wait()

        zero = jnp.zeros((LANES,), jnp.float32)

        def zrow(r, _):
            for cc in range(HIDDEN // LANES):
                acc[b][r, pl.ds(cc * LANES, LANES)] = zero
            return 0

        lax.fori_loop(0, CHUNK, zrow, 0)

        for i in range(NGRP):
            pltpu.sync_copy(
                xoff_hbm.at[pl.ds(i * n_rows + base, CHUNK)],
                idx[b].at[pl.ds(i * CHUNK, CHUNK)])
        for d in _gather_descs(tbig_hbm, tbl8_hbm, idx[b], acc[b], gsem[b]):
            d.start(add=True)

    for b in range(NBUF - 1):
        @pl.when(b < k_lim)
        def _(b=b):
            issue(b, b)

    def group_body(g, _):
        for j in range(NBUF):
            k = NBUF * g + j

            @pl.when(k < k_lim)
            def _(k=k, j=j):
                b = j
                c = wid + NUM_WORKERS * k
                base = c * CHUNK
                for d in _gather_descs(tbig_hbm, tbl8_hbm, idx[b], acc[b],
                                       gsem[b]):
                    d.wait()
                pltpu.make_async_copy(
                    acc[b], out_hbm.at[pl.ds(base, CHUNK)], osem[b]).start()

                @pl.when(k + NBUF - 1 < k_lim)
                def _():
                    issue(k + NBUF - 1, (j + NBUF - 1) % NBUF)

        return 0

    lax.fori_loop(0, (k_lim_cap + NBUF - 1) // NBUF, group_body, 0)

    for b in range(NBUF):
        @pl.when(k_lim > b)
        def _(b=b):
            pltpu.make_async_copy(
                acc[b], out_hbm.at[pl.ds(0, CHUNK)], osem[b]).wait()


def kernel(x, tables):
    n_rows = x.shape[0]
    n_chunks = n_rows // CHUNK
    assert n_chunks * CHUNK == n_rows
    k_lim_cap = -(-n_chunks // NUM_WORKERS)

    tbig = _build_pair_tables(tables)
    tbl8 = tables[2 * NPAIR]

    xi = x.astype(jnp.int32)
    pair_idx = (VOCAB * xi[:, 0:2 * NPAIR:2] + xi[:, 1:2 * NPAIR:2]
                + jnp.arange(NPAIR, dtype=jnp.int32)[None, :] * (VOCAB * VOCAB))
    xoff = jnp.concatenate([pair_idx.T, xi[:, 2 * NPAIR][None, :]], axis=0)
    xoff = xoff.reshape(-1)

    mesh = plsc.VectorSubcoreMesh(core_axis_name="c", subcore_axis_name="s")
    body = functools.partial(_sc_body, n_rows, k_lim_cap)
    run = pl.kernel(
        body,
        mesh=mesh,
        out_type=jax.ShapeDtypeStruct((n_rows, HIDDEN), jnp.float32),
        scratch_types=(
            [pltpu.VMEM((IDX_PER_CHUNK,), jnp.int32)] * NBUF
            + [pltpu.VMEM((CHUNK, HIDDEN), jnp.float32)] * NBUF
            + [pltpu.SemaphoreType.DMA] * (2 * NBUF)
        ),
    )
    return run(xoff, tbig, tbl8)
